# Initial kernel scaffold; baseline (speedup 1.0000x reference)
import jax
import jax.numpy as jnp
from jax.experimental import pallas as pl


def kernel(input1, input2):
    raise NotImplementedError("write your pallas kernel here")


# speedup vs baseline: 14.3881x; 14.3881x over previous
import jax
import jax.numpy as jnp
from jax.experimental import pallas as pl
from jax.experimental.pallas import tpu as pltpu

_B, _C, _H, _W = 4, 64, 256, 512
_HB = 8


def _resample_body(x2_ref, x1_ref, o_ref):
    disp = x2_ref[0, 0, :, :]
    iota_w = jax.lax.broadcasted_iota(
        jnp.int32, (_HB, _W), 1).astype(jnp.float32)
    x = iota_w + disp
    valid = (x >= 0.0) & (x <= float(_W - 1))
    xa = jnp.where(valid, x, -2.0)
    col = jax.lax.broadcasted_iota(
        jnp.int32, (_W, _W), 0).astype(jnp.float32)
    for hi in range(_HB):
        xr = xa[hi:hi + 1, :]
        s = jnp.maximum(1.0 - jnp.abs(col - xr), 0.0)
        sb = s.astype(jnp.bfloat16)
        lhs = x1_ref[0, :, 0, hi, :].astype(jnp.bfloat16)
        o_ref[0, :, 0, hi, :] = jnp.dot(
            lhs, sb, preferred_element_type=jnp.float32)


def kernel(input1, input2):
    b, c, h, w = input1.shape
    x1 = input1.reshape(b, c, h // _HB, _HB, w)
    x2 = input2.reshape(b, h // _HB, _HB, w)
    out = pl.pallas_call(
        _resample_body,
        grid=(b, h // _HB),
        in_specs=[
            pl.BlockSpec((1, 1, _HB, w), lambda bi, hb: (bi, hb, 0, 0)),
            pl.BlockSpec((1, c, 1, _HB, w), lambda bi, hb: (bi, 0, hb, 0, 0)),
        ],
        out_specs=pl.BlockSpec(
            (1, c, 1, _HB, w), lambda bi, hb: (bi, 0, hb, 0, 0)),
        out_shape=jax.ShapeDtypeStruct((b, c, h // _HB, _HB, w), jnp.float32),
        compiler_params=pltpu.CompilerParams(
            dimension_semantics=("parallel", "arbitrary"),
            vmem_limit_bytes=56 * 1024 * 1024,
        ),
    )(x2, x1)
    return out.reshape(b, c, h, w)
